# Initial kernel scaffold; baseline (speedup 1.0000x reference)
import functools

import jax
import jax.numpy as jnp
from jax import lax
from jax.experimental import pallas as pl
from jax.experimental.pallas import tpu as pltpu
from jax.experimental.pallas import tpu_sc as plsc

_B, _C, _L = 4, 8, 4096
_PATCH = 16
_HID = 768
_E = 64
_P = _L // _PATCH
_T = _B * _C * _P
_A = 2 * _T

_BM = 128
_TM = 256
_NTILE = _A // _TM
_S = _NTILE + _E - 1

_NC, _NS, _LANE = 2, 16, 16
_NW = _NC * _NS
_APW = _A // _NW
_TPW = _T // _NW
_CHUNK = 512
_NCHUNK = _A // _CHUNK
_GROWS = 64


def _router_body(xp_ref, wpt_ref, b_ref, wrt_ref,
                 tok_ref, topi_ref, topw_ref, rank_ref, cnt_ref, carry_ref):
    i = pl.program_id(0)

    @pl.when(i == 0)
    def _():
        carry_ref[...] = jnp.zeros_like(carry_ref)

    t = jnp.dot(xp_ref[...], wpt_ref[...],
                preferred_element_type=jnp.float32) + b_ref[...]
    tok_ref[...] = t
    logits = jnp.dot(t, wrt_ref[...], preferred_element_type=jnp.float32)
    m = jnp.max(logits, axis=1, keepdims=True)
    ex = jnp.exp(logits - m)
    p = ex / jnp.sum(ex, axis=1, keepdims=True)

    ids = lax.broadcasted_iota(jnp.int32, (_BM, _E), 1)
    w1 = jnp.max(p, axis=1, keepdims=True)
    i1 = jnp.min(jnp.where(p == w1, ids, _E), axis=1, keepdims=True)
    oh1 = ids == i1
    p2 = jnp.where(oh1, -1.0, p)
    w2 = jnp.max(p2, axis=1, keepdims=True)
    i2 = jnp.min(jnp.where(p2 == w2, ids, _E), axis=1, keepdims=True)
    oh2 = ids == i2

    oh1f = oh1.astype(jnp.float32)
    oh2f = oh2.astype(jnp.float32)
    tri = (lax.broadcasted_iota(jnp.int32, (_BM, _BM), 0)
           >= lax.broadcasted_iota(jnp.int32, (_BM, _BM), 1)).astype(jnp.float32)
    c1 = jnp.dot(tri, oh1f, preferred_element_type=jnp.float32)
    c2 = jnp.dot(tri, oh2f, preferred_element_type=jnp.float32)

    carry = carry_ref[0:1, :]
    prior0 = carry + (c1 - oh1f) + (c2 - oh2f)
    prior1 = carry + c1 + (c2 - oh2f)
    r1 = jnp.sum(oh1f * prior0, axis=1, keepdims=True)
    r2 = jnp.sum(oh2f * prior1, axis=1, keepdims=True)

    topi_ref[...] = jnp.concatenate([i1, i2], axis=1)
    topw_ref[...] = jnp.concatenate([w1, w2], axis=1)
    rank_ref[...] = jnp.concatenate([r1, r2], axis=1).astype(jnp.int32)

    newc = carry_ref[...] + jnp.broadcast_to(
        c1[_BM - 1:_BM, :] + c2[_BM - 1:_BM, :], (8, _E))
    carry_ref[...] = newc
    cnt_ref[...] = newc.astype(jnp.int32)


def _router(xp, wpt, b2, wrt):
    nblk = _T // _BM
    return pl.pallas_call(
        _router_body,
        grid=(nblk,),
        in_specs=[
            pl.BlockSpec((_BM, _PATCH), lambda i: (i, 0)),
            pl.BlockSpec((_PATCH, _HID), lambda i: (0, 0)),
            pl.BlockSpec((1, _HID), lambda i: (0, 0)),
            pl.BlockSpec((_HID, _E), lambda i: (0, 0)),
        ],
        out_specs=[
            pl.BlockSpec((_BM, _HID), lambda i: (i, 0)),
            pl.BlockSpec((_BM, 2), lambda i: (i, 0)),
            pl.BlockSpec((_BM, 2), lambda i: (i, 0)),
            pl.BlockSpec((_BM, 2), lambda i: (i, 0)),
            pl.BlockSpec((8, _E), lambda i: (0, 0)),
        ],
        out_shape=[
            jax.ShapeDtypeStruct((_T, _HID), jnp.float32),
            jax.ShapeDtypeStruct((_T, 2), jnp.int32),
            jax.ShapeDtypeStruct((_T, 2), jnp.float32),
            jax.ShapeDtypeStruct((_T, 2), jnp.int32),
            jax.ShapeDtypeStruct((8, _E), jnp.int32),
        ],
        scratch_shapes=[pltpu.VMEM((8, _E), jnp.float32)],
    )(xp, wpt, b2, wrt)


def _dispatch_body(tok_hbm, ei_hbm, rk_hbm, wt_hbm, off_hbm,
                   xs_hbm, ws_hbm, dest_hbm,
                   off_v, e_buf, r_buf, w_buf, d_buf,
                   inv_loc, w_loc, rows_v, sem):
    wid = lax.axis_index("s") * _NC + lax.axis_index("c")
    j0 = wid * _APW

    pltpu.sync_copy(off_hbm, off_v)

    for c in range(_NCHUNK):
        pltpu.sync_copy(ei_hbm.at[pl.ds(c * _CHUNK, _CHUNK)], e_buf)
        pltpu.sync_copy(rk_hbm.at[pl.ds(c * _CHUNK, _CHUNK)], r_buf)
        pltpu.sync_copy(wt_hbm.at[pl.ds(c * _CHUNK, _CHUNK)], w_buf)

        def vstep(v, _, c=c):
            sl = pl.ds(v * _LANE, _LANE)
            e = e_buf[sl]
            base = plsc.load_gather(off_v, [e])
            d = base + r_buf[sl]
            d_buf[sl] = d
            gi = c * _CHUNK + v * _LANE + lax.iota(jnp.int32, (_LANE,))
            tok = lax.shift_right_logical(gi, 1)
            msk = (d >= j0) & (d < j0 + _APW)
            dl = jnp.where(msk, d - j0, 0)
            plsc.store_scatter(inv_loc, [dl], tok, mask=msk)
            plsc.store_scatter(w_loc, [dl], w_buf[sl], mask=msk)
            return 0

        lax.fori_loop(0, _CHUNK // _LANE, vstep, 0)

        @pl.when(wid == c)
        def _(c=c):
            pltpu.sync_copy(d_buf, dest_hbm.at[pl.ds(c * _CHUNK, _CHUNK)])

    pltpu.sync_copy(w_loc, ws_hbm.at[pl.ds(j0, _APW)])
    for c2 in range(_APW // _GROWS):
        idxs = inv_loc.at[pl.ds(c2 * _GROWS, _GROWS)]
        pltpu.async_copy(tok_hbm.at[idxs], rows_v, sem).wait()
        pltpu.sync_copy(rows_v, xs_hbm.at[pl.ds(j0 + c2 * _GROWS, _GROWS)])


def _dispatch(tokens, ei, rk, wt, offs):
    mesh = plsc.VectorSubcoreMesh(core_axis_name="c", subcore_axis_name="s")
    f = pl.kernel(
        _dispatch_body,
        out_type=[
            jax.ShapeDtypeStruct((_A, _HID), jnp.float32),
            jax.ShapeDtypeStruct((_A,), jnp.float32),
            jax.ShapeDtypeStruct((_A,), jnp.int32),
        ],
        mesh=mesh,
        scratch_types=[
            pltpu.VMEM((_E,), jnp.int32),
            pltpu.VMEM((_CHUNK,), jnp.int32),
            pltpu.VMEM((_CHUNK,), jnp.int32),
            pltpu.VMEM((_CHUNK,), jnp.float32),
            pltpu.VMEM((_CHUNK,), jnp.int32),
            pltpu.VMEM((_APW,), jnp.int32),
            pltpu.VMEM((_APW,), jnp.float32),
            pltpu.VMEM((_GROWS, _HID), jnp.float32),
            pltpu.SemaphoreType.DMA,
        ],
    )
    return f(tokens, ei, rk, wt, offs)


def _ffn_body(g_ref, t_ref, f_ref, o_ref, e_ref,
              xs_ref, ws_ref, wg_ref, wu_ref, wd_ref, out_ref):
    s = pl.program_id(0)
    g = g_ref[s]
    t = t_ref[s]

    x = xs_ref[...]
    nt = (((1,), (1,)), ((), ()))
    gt = lax.dot_general(x, wg_ref[0], nt, preferred_element_type=jnp.float32)
    ut = lax.dot_general(x, wu_ref[0], nt, preferred_element_type=jnp.float32)
    h = (gt * lax.logistic(gt)) * ut
    h = h * ws_ref[...]
    rows = t * _TM + lax.broadcasted_iota(jnp.int32, (_TM, 1), 0)
    msk = (rows >= o_ref[g]) & (rows < e_ref[g])
    h = jnp.where(msk, h, 0.0)
    o = lax.dot_general(h, wd_ref[0], nt, preferred_element_type=jnp.float32)

    @pl.when(f_ref[s] == 1)
    def _():
        out_ref[...] = o

    @pl.when(f_ref[s] == 0)
    def _():
        out_ref[...] += o


def _ffn(g_of, t_of, firsts, offs, ends, xs, ws2, Wgate, Wup, Wdown):
    grid_spec = pltpu.PrefetchScalarGridSpec(
        num_scalar_prefetch=5,
        grid=(_S,),
        in_specs=[
            pl.BlockSpec((_TM, _HID), lambda i, g, t, f, o, e: (t[i], 0)),
            pl.BlockSpec((_TM, 1), lambda i, g, t, f, o, e: (t[i], 0)),
            pl.BlockSpec((1, _HID, _HID), lambda i, g, t, f, o, e: (g[i], 0, 0)),
            pl.BlockSpec((1, _HID, _HID), lambda i, g, t, f, o, e: (g[i], 0, 0)),
            pl.BlockSpec((1, _HID, _HID), lambda i, g, t, f, o, e: (g[i], 0, 0)),
        ],
        out_specs=pl.BlockSpec((_TM, _HID), lambda i, g, t, f, o, e: (t[i], 0)),
    )
    return pl.pallas_call(
        _ffn_body,
        grid_spec=grid_spec,
        out_shape=jax.ShapeDtypeStruct((_A, _HID), jnp.float32),
    )(g_of, t_of, firsts, offs, ends, xs, ws2, Wgate, Wup, Wdown)


def _combine_body(h_hbm, d_hbm, out_hbm, d_buf, rows_v, out_buf, sem):
    wid = lax.axis_index("s") * _NC + lax.axis_index("c")
    a0 = wid * _APW
    t0 = wid * _TPW

    pltpu.sync_copy(d_hbm.at[pl.ds(a0, _APW)], d_buf)
    for c in range(_APW // _GROWS):
        idxs = d_buf.at[pl.ds(c * _GROWS, _GROWS)]
        pltpu.async_copy(h_hbm.at[idxs], rows_v, sem).wait()

        def rstep(r, _):
            def vstep(v, __):
                sl = pl.ds(v * _LANE, _LANE)
                out_buf[r, sl] = rows_v[2 * r, sl] + rows_v[2 * r + 1, sl]
                return 0
            return lax.fori_loop(0, _HID // _LANE, vstep, 0)

        lax.fori_loop(0, _GROWS // 2, rstep, 0)
        pltpu.sync_copy(out_buf, out_hbm.at[pl.ds(t0 + c * (_GROWS // 2),
                                                  _GROWS // 2)])


def _combine(h, dest):
    mesh = plsc.VectorSubcoreMesh(core_axis_name="c", subcore_axis_name="s")
    f = pl.kernel(
        _combine_body,
        out_type=jax.ShapeDtypeStruct((_T, _HID), jnp.float32),
        mesh=mesh,
        scratch_types=[
            pltpu.VMEM((_APW,), jnp.int32),
            pltpu.VMEM((_GROWS, _HID), jnp.float32),
            pltpu.VMEM((_GROWS // 2, _HID), jnp.float32),
            pltpu.SemaphoreType.DMA,
        ],
    )
    return f(h, dest)


def kernel(x, Wproj, bproj, Wrouter, Wgate, Wup, Wdown):
    xp = x.reshape(_T, _PATCH)
    tokens, topi, topw, rank, counts8 = _router(
        xp, Wproj.T, bproj.reshape(1, _HID), Wrouter.T)

    counts = counts8[0]
    csum = jnp.cumsum(counts)
    offs = (csum - counts).astype(jnp.int32)
    ends = csum.astype(jnp.int32)

    ei = topi.reshape(_A)
    rk = rank.reshape(_A)
    wt = topw.reshape(_A)
    xs, ws, dest = _dispatch(tokens, ei, rk, wt, offs)

    t_start = offs // _TM
    t_end = (ends + _TM - 1) // _TM
    visits = jnp.where(counts > 0, t_end - t_start, 0)
    cumv = jnp.cumsum(visits)
    first_step = cumv - visits
    v_tot = cumv[-1]
    steps = jnp.arange(_S, dtype=jnp.int32)
    s_eff = jnp.minimum(steps, v_tot - 1)
    g_of = jnp.sum((s_eff[:, None] >= cumv[None, :]).astype(jnp.int32),
                   axis=1)
    t_of = (t_start[g_of] + (s_eff - first_step[g_of])).astype(jnp.int32)
    firsts = jnp.concatenate([
        jnp.ones((1,), jnp.int32),
        (t_of[1:] != t_of[:-1]).astype(jnp.int32),
    ])

    h = _ffn(g_of.astype(jnp.int32), t_of, firsts, offs, ends,
             xs, ws.reshape(_A, 1), Wgate, Wup, Wdown)
    outf = _combine(h, dest)
    return outf.reshape(_B, _C * _P, _HID)


# speedup vs baseline: 6.1997x; 6.1997x over previous
import functools

import jax
import jax.numpy as jnp
from jax import lax
from jax.experimental import pallas as pl
from jax.experimental.pallas import tpu as pltpu
from jax.experimental.pallas import tpu_sc as plsc

_B, _C, _L = 4, 8, 4096
_PATCH = 16
_HID = 768
_E = 64
_P = _L // _PATCH
_T = _B * _C * _P
_A = 2 * _T

_BM = 128
_TM = 256
_NTILE = _A // _TM
_S = _NTILE + _E - 1

_NC, _NS, _LANE = 2, 16, 16
_NW = _NC * _NS
_APW = _A // _NW
_TPW = _T // _NW
_CHUNK = 512
_NCHUNK = _A // _CHUNK
_GROWS = 64


def _router_body(xp_ref, wpt_ref, b_ref, wrt_ref,
                 tok_ref, topi_ref, topw_ref, rank_ref, cnt_ref, carry_ref):
    i = pl.program_id(0)

    @pl.when(i == 0)
    def _():
        carry_ref[...] = jnp.zeros_like(carry_ref)

    t = jnp.dot(xp_ref[...], wpt_ref[...],
                preferred_element_type=jnp.float32) + b_ref[...]
    tok_ref[...] = t
    logits = jnp.dot(t, wrt_ref[...], preferred_element_type=jnp.float32)
    m = jnp.max(logits, axis=1, keepdims=True)
    ex = jnp.exp(logits - m)
    p = ex / jnp.sum(ex, axis=1, keepdims=True)

    ids = lax.broadcasted_iota(jnp.int32, (_BM, _E), 1)
    w1 = jnp.max(p, axis=1, keepdims=True)
    i1 = jnp.min(jnp.where(p == w1, ids, _E), axis=1, keepdims=True)
    oh1 = ids == i1
    p2 = jnp.where(oh1, -1.0, p)
    w2 = jnp.max(p2, axis=1, keepdims=True)
    i2 = jnp.min(jnp.where(p2 == w2, ids, _E), axis=1, keepdims=True)
    oh2 = ids == i2

    oh1f = oh1.astype(jnp.float32)
    oh2f = oh2.astype(jnp.float32)
    tri = (lax.broadcasted_iota(jnp.int32, (_BM, _BM), 0)
           >= lax.broadcasted_iota(jnp.int32, (_BM, _BM), 1)).astype(jnp.float32)
    c1 = jnp.dot(tri, oh1f, preferred_element_type=jnp.float32)
    c2 = jnp.dot(tri, oh2f, preferred_element_type=jnp.float32)

    carry = carry_ref[0:1, :]
    prior0 = carry + (c1 - oh1f) + (c2 - oh2f)
    prior1 = carry + c1 + (c2 - oh2f)
    r1 = jnp.sum(oh1f * prior0, axis=1, keepdims=True)
    r2 = jnp.sum(oh2f * prior1, axis=1, keepdims=True)

    topi_ref[...] = jnp.concatenate([i1, i2], axis=1)
    topw_ref[...] = jnp.concatenate([w1, w2], axis=1)
    rank_ref[...] = jnp.concatenate([r1, r2], axis=1).astype(jnp.int32)

    newc = carry_ref[...] + jnp.broadcast_to(
        c1[_BM - 1:_BM, :] + c2[_BM - 1:_BM, :], (8, _E))
    carry_ref[...] = newc
    cnt_ref[...] = newc.astype(jnp.int32)


def _router(xp, wpt, b2, wrt):
    nblk = _T // _BM
    return pl.pallas_call(
        _router_body,
        grid=(nblk,),
        in_specs=[
            pl.BlockSpec((_BM, _PATCH), lambda i: (i, 0)),
            pl.BlockSpec((_PATCH, _HID), lambda i: (0, 0)),
            pl.BlockSpec((1, _HID), lambda i: (0, 0)),
            pl.BlockSpec((_HID, _E), lambda i: (0, 0)),
        ],
        out_specs=[
            pl.BlockSpec((_BM, _HID), lambda i: (i, 0)),
            pl.BlockSpec((_BM, 2), lambda i: (i, 0)),
            pl.BlockSpec((_BM, 2), lambda i: (i, 0)),
            pl.BlockSpec((_BM, 2), lambda i: (i, 0)),
            pl.BlockSpec((8, _E), lambda i: (0, 0)),
        ],
        out_shape=[
            jax.ShapeDtypeStruct((_T, _HID), jnp.float32),
            jax.ShapeDtypeStruct((_T, 2), jnp.int32),
            jax.ShapeDtypeStruct((_T, 2), jnp.float32),
            jax.ShapeDtypeStruct((_T, 2), jnp.int32),
            jax.ShapeDtypeStruct((8, _E), jnp.int32),
        ],
        scratch_shapes=[pltpu.VMEM((8, _E), jnp.float32)],
    )(xp, wpt, b2, wrt)


def _dispatch_body(tok_hbm, ei_hbm, rk_hbm, wt_hbm, off_hbm,
                   xs_hbm, ws_hbm, dest_hbm,
                   off_v, e_buf, r_buf, w_buf, d_buf,
                   inv_loc, w_loc, rows_v, sem):
    wid = lax.axis_index("s") * _NC + lax.axis_index("c")
    j0 = wid * _APW

    pltpu.sync_copy(off_hbm, off_v)

    for c in range(_NCHUNK):
        pltpu.sync_copy(ei_hbm.at[pl.ds(c * _CHUNK, _CHUNK)], e_buf)
        pltpu.sync_copy(rk_hbm.at[pl.ds(c * _CHUNK, _CHUNK)], r_buf)
        pltpu.sync_copy(wt_hbm.at[pl.ds(c * _CHUNK, _CHUNK)], w_buf)

        def vstep(v, _, c=c):
            sl = pl.ds(v * _LANE, _LANE)
            e = e_buf[sl]
            base = plsc.load_gather(off_v, [e])
            d = base + r_buf[sl]
            d_buf[sl] = d
            gi = c * _CHUNK + v * _LANE + lax.iota(jnp.int32, _LANE)
            tok = lax.shift_right_logical(gi, 1)
            msk = (d >= j0) & (d < j0 + _APW)
            dl = jnp.where(msk, d - j0, 0)
            plsc.store_scatter(inv_loc, [dl], tok, mask=msk)
            plsc.store_scatter(w_loc, [dl], w_buf[sl], mask=msk)
            return 0

        lax.fori_loop(0, _CHUNK // _LANE, vstep, 0)

        @pl.when(wid == c)
        def _(c=c):
            pltpu.sync_copy(d_buf, dest_hbm.at[pl.ds(c * _CHUNK, _CHUNK)])

    pltpu.sync_copy(w_loc, ws_hbm.at[pl.ds(j0, _APW)])
    for c2 in range(_APW // _GROWS):
        idxs = inv_loc.at[pl.ds(c2 * _GROWS, _GROWS)]
        pltpu.async_copy(tok_hbm.at[idxs], rows_v, sem).wait()
        pltpu.sync_copy(rows_v, xs_hbm.at[pl.ds(j0 + c2 * _GROWS, _GROWS)])


def _dispatch(tokens, ei, rk, wt, offs):
    mesh = plsc.VectorSubcoreMesh(core_axis_name="c", subcore_axis_name="s")
    f = pl.kernel(
        _dispatch_body,
        out_type=[
            jax.ShapeDtypeStruct((_A, _HID), jnp.float32),
            jax.ShapeDtypeStruct((_A,), jnp.float32),
            jax.ShapeDtypeStruct((_A,), jnp.int32),
        ],
        mesh=mesh,
        scratch_types=[
            pltpu.VMEM((_E,), jnp.int32),
            pltpu.VMEM((_CHUNK,), jnp.int32),
            pltpu.VMEM((_CHUNK,), jnp.int32),
            pltpu.VMEM((_CHUNK,), jnp.float32),
            pltpu.VMEM((_CHUNK,), jnp.int32),
            pltpu.VMEM((_APW,), jnp.int32),
            pltpu.VMEM((_APW,), jnp.float32),
            pltpu.VMEM((_GROWS, _HID), jnp.float32),
            pltpu.SemaphoreType.DMA,
        ],
        compiler_params=pltpu.CompilerParams(needs_layout_passes=False),
    )
    return f(tokens, ei, rk, wt, offs)


def _ffn_body(g_ref, t_ref, f_ref, o_ref, e_ref, v_ref,
              xs_ref, ws_ref, wg_ref, wu_ref, wd_ref, out_ref):
    s = pl.program_id(0)
    g = g_ref[s]
    t = t_ref[s]

    x = xs_ref[...]
    nt = (((1,), (1,)), ((), ()))
    gt = lax.dot_general(x, wg_ref[0], nt, preferred_element_type=jnp.float32)
    ut = lax.dot_general(x, wu_ref[0], nt, preferred_element_type=jnp.float32)
    h = (gt * lax.logistic(gt)) * ut
    h = h * ws_ref[...]
    rows = t * _TM + lax.broadcasted_iota(jnp.int32, (_TM, 1), 0)
    msk = (rows >= o_ref[g]) & (rows < e_ref[g]) & (v_ref[s] == 1)
    h = jnp.where(msk, h, 0.0)
    o = lax.dot_general(h, wd_ref[0], nt, preferred_element_type=jnp.float32)

    @pl.when(f_ref[s] == 1)
    def _():
        out_ref[...] = o

    @pl.when(f_ref[s] == 0)
    def _():
        out_ref[...] += o


def _ffn(g_of, t_of, firsts, offs, ends, valid, xs, ws2, Wgate, Wup, Wdown):
    grid_spec = pltpu.PrefetchScalarGridSpec(
        num_scalar_prefetch=6,
        grid=(_S,),
        in_specs=[
            pl.BlockSpec((_TM, _HID), lambda i, g, t, f, o, e, v: (t[i], 0)),
            pl.BlockSpec((_TM, 1), lambda i, g, t, f, o, e, v: (t[i], 0)),
            pl.BlockSpec((1, _HID, _HID),
                         lambda i, g, t, f, o, e, v: (g[i], 0, 0)),
            pl.BlockSpec((1, _HID, _HID),
                         lambda i, g, t, f, o, e, v: (g[i], 0, 0)),
            pl.BlockSpec((1, _HID, _HID),
                         lambda i, g, t, f, o, e, v: (g[i], 0, 0)),
        ],
        out_specs=pl.BlockSpec((_TM, _HID),
                               lambda i, g, t, f, o, e, v: (t[i], 0)),
    )
    return pl.pallas_call(
        _ffn_body,
        grid_spec=grid_spec,
        out_shape=jax.ShapeDtypeStruct((_A, _HID), jnp.float32),
    )(g_of, t_of, firsts, offs, ends, valid, xs, ws2, Wgate, Wup, Wdown)


def _combine_body(h_hbm, d_hbm, out_hbm, d_buf, rows_v, out_buf, sem):
    wid = lax.axis_index("s") * _NC + lax.axis_index("c")
    a0 = wid * _APW
    t0 = wid * _TPW

    pltpu.sync_copy(d_hbm.at[pl.ds(a0, _APW)], d_buf)
    for c in range(_APW // _GROWS):
        idxs = d_buf.at[pl.ds(c * _GROWS, _GROWS)]
        pltpu.async_copy(h_hbm.at[idxs], rows_v, sem).wait()

        def rstep(r, _):
            def vstep(v, __):
                sl = pl.ds(v * _LANE, _LANE)
                out_buf[r, sl] = rows_v[2 * r, sl] + rows_v[2 * r + 1, sl]
                return 0
            return lax.fori_loop(0, _HID // _LANE, vstep, 0)

        lax.fori_loop(0, _GROWS // 2, rstep, 0)
        pltpu.sync_copy(out_buf, out_hbm.at[pl.ds(t0 + c * (_GROWS // 2),
                                                  _GROWS // 2)])


def _combine(h, dest):
    mesh = plsc.VectorSubcoreMesh(core_axis_name="c", subcore_axis_name="s")
    f = pl.kernel(
        _combine_body,
        out_type=jax.ShapeDtypeStruct((_T, _HID), jnp.float32),
        mesh=mesh,
        scratch_types=[
            pltpu.VMEM((_APW,), jnp.int32),
            pltpu.VMEM((_GROWS, _HID), jnp.float32),
            pltpu.VMEM((_GROWS // 2, _HID), jnp.float32),
            pltpu.SemaphoreType.DMA,
        ],
        compiler_params=pltpu.CompilerParams(needs_layout_passes=False),
    )
    return f(h, dest)


def kernel(x, Wproj, bproj, Wrouter, Wgate, Wup, Wdown):
    xp = x.reshape(_T, _PATCH)
    tokens, topi, topw, rank, counts8 = _router(
        xp, Wproj.T, bproj.reshape(1, _HID), Wrouter.T)

    counts = counts8[0]
    csum = jnp.cumsum(counts)
    offs = (csum - counts).astype(jnp.int32)
    ends = csum.astype(jnp.int32)

    ei = topi.reshape(_A)
    rk = rank.reshape(_A)
    wt = topw.reshape(_A)
    xs, ws, dest = _dispatch(tokens, ei, rk, wt, offs)

    t_start = offs // _TM
    t_end = (ends + _TM - 1) // _TM
    visits = jnp.where(counts > 0, t_end - t_start, 0)
    cumv = jnp.cumsum(visits)
    first_step = cumv - visits
    v_tot = cumv[-1]
    steps = jnp.arange(_S, dtype=jnp.int32)
    s_eff = jnp.minimum(steps, v_tot - 1)
    g_of = jnp.sum((s_eff[:, None] >= cumv[None, :]).astype(jnp.int32),
                   axis=1)
    t_of = (t_start[g_of] + (s_eff - first_step[g_of])).astype(jnp.int32)
    firsts = jnp.concatenate([
        jnp.ones((1,), jnp.int32),
        (t_of[1:] != t_of[:-1]).astype(jnp.int32),
    ])
    valid = (steps < v_tot).astype(jnp.int32)

    h = _ffn(g_of.astype(jnp.int32), t_of, firsts, offs, ends, valid,
             xs, ws.reshape(_A, 1), Wgate, Wup, Wdown)
    outf = _combine(h, dest)
    return outf.reshape(_B, _C * _P, _HID)


# speedup vs baseline: 8.4934x; 1.3700x over previous
import jax
import jax.numpy as jnp
from jax import lax
from jax.experimental import pallas as pl
from jax.experimental.pallas import tpu as pltpu
from jax.experimental.pallas import tpu_sc as plsc

_B, _C, _L = 4, 8, 4096
_PATCH = 16
_HID = 768
_E = 64
_P = _L // _PATCH
_T = _B * _C * _P
_A = 2 * _T

_BM = 256
_TM = 256
_NTILE = _A // _TM
_S = _NTILE + _E - 1

_NC, _NS, _LANE = 2, 16, 16
_NW = _NC * _NS
_APW = _A // _NW
_TPW = _T // _NW
_DCH = 64
_CCH = 32


def _router_body(xp_ref, wpt_ref, b_ref, wrt_ref,
                 tok_ref, topi_ref, topw_ref, rank_ref, cnt_ref, carry_ref):
    i = pl.program_id(0)

    @pl.when(i == 0)
    def _():
        carry_ref[...] = jnp.zeros_like(carry_ref)

    t = jnp.dot(xp_ref[...], wpt_ref[...],
                preferred_element_type=jnp.float32) + b_ref[...]
    tok_ref[...] = t
    logits = jnp.dot(t, wrt_ref[...], preferred_element_type=jnp.float32)
    m = jnp.max(logits, axis=1, keepdims=True)
    ex = jnp.exp(logits - m)
    p = ex / jnp.sum(ex, axis=1, keepdims=True)

    ids = lax.broadcasted_iota(jnp.int32, (_BM, _E), 1)
    w1 = jnp.max(p, axis=1, keepdims=True)
    i1 = jnp.min(jnp.where(p == w1, ids, _E), axis=1, keepdims=True)
    oh1 = ids == i1
    p2 = jnp.where(oh1, -1.0, p)
    w2 = jnp.max(p2, axis=1, keepdims=True)
    i2 = jnp.min(jnp.where(p2 == w2, ids, _E), axis=1, keepdims=True)
    oh2 = ids == i2

    oh1f = oh1.astype(jnp.float32)
    oh2f = oh2.astype(jnp.float32)
    tri = (lax.broadcasted_iota(jnp.int32, (_BM, _BM), 0)
           >= lax.broadcasted_iota(jnp.int32, (_BM, _BM), 1)).astype(jnp.float32)
    c1 = jnp.dot(tri, oh1f, preferred_element_type=jnp.float32)
    c2 = jnp.dot(tri, oh2f, preferred_element_type=jnp.float32)

    carry = carry_ref[0:1, :]
    prior0 = carry + (c1 - oh1f) + (c2 - oh2f)
    prior1 = carry + c1 + (c2 - oh2f)
    r1 = jnp.sum(oh1f * prior0, axis=1, keepdims=True)
    r2 = jnp.sum(oh2f * prior1, axis=1, keepdims=True)

    topi_ref[...] = jnp.concatenate([i1, i2], axis=1)
    topw_ref[...] = jnp.concatenate([w1, w2], axis=1)
    rank_ref[...] = jnp.concatenate([r1, r2], axis=1).astype(jnp.int32)

    newc = carry_ref[...] + jnp.broadcast_to(
        c1[_BM - 1:_BM, :] + c2[_BM - 1:_BM, :], (8, _E))
    carry_ref[...] = newc
    cnt_ref[...] = newc.astype(jnp.int32)


def _router(xp, wpt, b2, wrt):
    nblk = _T // _BM
    return pl.pallas_call(
        _router_body,
        grid=(nblk,),
        in_specs=[
            pl.BlockSpec((_BM, _PATCH), lambda i: (i, 0)),
            pl.BlockSpec((_PATCH, _HID), lambda i: (0, 0)),
            pl.BlockSpec((1, _HID), lambda i: (0, 0)),
            pl.BlockSpec((_HID, _E), lambda i: (0, 0)),
        ],
        out_specs=[
            pl.BlockSpec((_BM, _HID), lambda i: (i, 0)),
            pl.BlockSpec((_BM, 2), lambda i: (i, 0)),
            pl.BlockSpec((_BM, 2), lambda i: (i, 0)),
            pl.BlockSpec((_BM, 2), lambda i: (i, 0)),
            pl.BlockSpec((8, _E), lambda i: (0, 0)),
        ],
        out_shape=[
            jax.ShapeDtypeStruct((_T, _HID), jnp.float32),
            jax.ShapeDtypeStruct((_T, 2), jnp.int32),
            jax.ShapeDtypeStruct((_T, 2), jnp.float32),
            jax.ShapeDtypeStruct((_T, 2), jnp.int32),
            jax.ShapeDtypeStruct((8, _E), jnp.int32),
        ],
        scratch_shapes=[pltpu.VMEM((8, _E), jnp.float32)],
    )(xp, wpt, b2, wrt)


def _dispatch_body(tok_hbm, ei_hbm, rk_hbm, off_hbm,
                   xs_hbm, d0_hbm, d1_hbm,
                   off_v, e_buf, r_buf, d0_buf, d1_buf,
                   tr_a, tr_b, gsem, s0sem, s1sem):
    wid = lax.axis_index("s") * _NC + lax.axis_index("c")
    t0 = wid * _TPW
    a0 = wid * _APW

    pltpu.sync_copy(off_hbm, off_v)
    pltpu.sync_copy(ei_hbm.at[pl.ds(a0, _APW)], e_buf)
    pltpu.sync_copy(rk_hbm.at[pl.ds(a0, _APW)], r_buf)

    nch = _TPW // _DCH
    for c in range(nch):
        for l in range(_DCH // _LANE):
            even = 2 * (c * _DCH + l * _LANE + lax.iota(jnp.int32, _LANE))
            e0 = plsc.load_gather(e_buf, [even])
            r0 = plsc.load_gather(r_buf, [even])
            d0 = plsc.load_gather(off_v, [e0]) + r0
            d0_buf[c, pl.ds(l * _LANE, _LANE)] = d0
            e1 = plsc.load_gather(e_buf, [even + 1])
            r1 = plsc.load_gather(r_buf, [even + 1])
            d1 = plsc.load_gather(off_v, [e1]) + r1
            d1_buf[c, pl.ds(l * _LANE, _LANE)] = d1

    pltpu.sync_copy(d0_buf, d0_hbm.at[wid])
    pltpu.sync_copy(d1_buf, d1_hbm.at[wid])

    bufs = (tr_a, tr_b)
    rd = pltpu.async_copy(tok_hbm.at[pl.ds(t0, _DCH)], tr_a, gsem)
    sc0 = sc1 = None
    for c in range(nch):
        buf = bufs[c % 2]
        rd.wait()
        if sc0 is not None:
            sc0.wait()
            sc1.wait()
        if c < nch - 1:
            rd = pltpu.async_copy(
                tok_hbm.at[pl.ds(t0 + (c + 1) * _DCH, _DCH)],
                bufs[(c + 1) % 2], gsem)
        sc0 = pltpu.async_copy(buf, xs_hbm.at[d0_buf.at[c]], s0sem)
        sc1 = pltpu.async_copy(buf, xs_hbm.at[d1_buf.at[c]], s1sem)
    sc0.wait()
    sc1.wait()


def _dispatch(tokens, ei, rk, offs):
    mesh = plsc.VectorSubcoreMesh(core_axis_name="c", subcore_axis_name="s")
    nch = _TPW // _DCH
    f = pl.kernel(
        _dispatch_body,
        out_type=[
            jax.ShapeDtypeStruct((_A, _HID), jnp.float32),
            jax.ShapeDtypeStruct((_NW, nch, _DCH), jnp.int32),
            jax.ShapeDtypeStruct((_NW, nch, _DCH), jnp.int32),
        ],
        mesh=mesh,
        scratch_types=[
            pltpu.VMEM((_E,), jnp.int32),
            pltpu.VMEM((_APW,), jnp.int32),
            pltpu.VMEM((_APW,), jnp.int32),
            pltpu.VMEM((nch, _DCH), jnp.int32),
            pltpu.VMEM((nch, _DCH), jnp.int32),
            pltpu.VMEM((_DCH, _HID), jnp.float32),
            pltpu.VMEM((_DCH, _HID), jnp.float32),
            pltpu.SemaphoreType.DMA,
            pltpu.SemaphoreType.DMA,
            pltpu.SemaphoreType.DMA,
        ],
        compiler_params=pltpu.CompilerParams(needs_layout_passes=False),
    )
    return f(tokens, ei, rk, offs)


def _ffn_body(g_ref, t_ref, f_ref, o_ref, e_ref, v_ref,
              xs_ref, wg_ref, wu_ref, wd_ref, out_ref):
    s = pl.program_id(0)
    g = g_ref[s]
    t = t_ref[s]

    x = xs_ref[...]
    nt = (((1,), (1,)), ((), ()))
    gt = lax.dot_general(x, wg_ref[0], nt, preferred_element_type=jnp.float32)
    ut = lax.dot_general(x, wu_ref[0], nt, preferred_element_type=jnp.float32)
    h = (gt * lax.logistic(gt)) * ut
    rows = t * _TM + lax.broadcasted_iota(jnp.int32, (_TM, 1), 0)
    msk = (rows >= o_ref[g]) & (rows < e_ref[g]) & (v_ref[s] == 1)
    h = jnp.where(msk, h, 0.0)
    o = lax.dot_general(h, wd_ref[0], nt, preferred_element_type=jnp.float32)

    @pl.when(f_ref[s] == 1)
    def _():
        out_ref[...] = o

    @pl.when(f_ref[s] == 0)
    def _():
        out_ref[...] += o


def _ffn(g_of, t_of, firsts, offs, ends, valid, xs, Wgate, Wup, Wdown):
    grid_spec = pltpu.PrefetchScalarGridSpec(
        num_scalar_prefetch=6,
        grid=(_S,),
        in_specs=[
            pl.BlockSpec((_TM, _HID), lambda i, g, t, f, o, e, v: (t[i], 0)),
            pl.BlockSpec((1, _HID, _HID),
                         lambda i, g, t, f, o, e, v: (g[i], 0, 0)),
            pl.BlockSpec((1, _HID, _HID),
                         lambda i, g, t, f, o, e, v: (g[i], 0, 0)),
            pl.BlockSpec((1, _HID, _HID),
                         lambda i, g, t, f, o, e, v: (g[i], 0, 0)),
        ],
        out_specs=pl.BlockSpec((_TM, _HID),
                               lambda i, g, t, f, o, e, v: (t[i], 0)),
    )
    return pl.pallas_call(
        _ffn_body,
        grid_spec=grid_spec,
        out_shape=jax.ShapeDtypeStruct((_A, _HID), jnp.float32),
    )(g_of, t_of, firsts, offs, ends, valid, xs, Wgate, Wup, Wdown)


def _combine_body(h_hbm, d0_hbm, d1_hbm, wt_hbm, out_hbm,
                  d0_buf, d1_buf, w_buf,
                  r0a, r0b, r1a, r1b, ob, g0sem, g1sem):
    wid = lax.axis_index("s") * _NC + lax.axis_index("c")
    t0 = wid * _TPW
    a0 = wid * _APW

    pltpu.sync_copy(d0_hbm.at[wid], d0_buf)
    pltpu.sync_copy(d1_hbm.at[wid], d1_buf)
    pltpu.sync_copy(wt_hbm.at[pl.ds(a0, _APW)], w_buf)

    nch = _TPW // _CCH
    r0bufs = (r0a, r0b)
    r1bufs = (r1a, r1b)

    def idx_slices(c):
        row = c * _CCH // _DCH
        off = (c * _CCH) % _DCH
        return (d0_buf.at[row, pl.ds(off, _CCH)],
                d1_buf.at[row, pl.ds(off, _CCH)])

    i0, i1 = idx_slices(0)
    g0 = pltpu.async_copy(h_hbm.at[i0], r0a, g0sem)
    g1 = pltpu.async_copy(h_hbm.at[i1], r1a, g1sem)
    for c in range(nch):
        r0 = r0bufs[c % 2]
        r1 = r1bufs[c % 2]
        g0.wait()
        g1.wait()
        if c < nch - 1:
            i0, i1 = idx_slices(c + 1)
            g0 = pltpu.async_copy(h_hbm.at[i0], r0bufs[(c + 1) % 2], g0sem)
            g1 = pltpu.async_copy(h_hbm.at[i1], r1bufs[(c + 1) % 2], g1sem)

        def rstep(r, _, c=c, r0=r0, r1=r1):
            w0 = plsc.load_gather(
                w_buf, [jnp.full((_LANE,), 2 * (c * _CCH + r), jnp.int32)])
            w1 = plsc.load_gather(
                w_buf, [jnp.full((_LANE,), 2 * (c * _CCH + r) + 1, jnp.int32)])

            def vstep(v, __):
                sl = pl.ds(v * _LANE, _LANE)
                ob[r, sl] = r0[r, sl] * w0 + r1[r, sl] * w1
                return 0
            return lax.fori_loop(0, _HID // _LANE, vstep, 0)

        lax.fori_loop(0, _CCH, rstep, 0)
        pltpu.sync_copy(ob, out_hbm.at[pl.ds(t0 + c * _CCH, _CCH)])


def _combine(h, d0, d1, wt):
    mesh = plsc.VectorSubcoreMesh(core_axis_name="c", subcore_axis_name="s")
    nch = _TPW // _DCH
    f = pl.kernel(
        _combine_body,
        out_type=jax.ShapeDtypeStruct((_T, _HID), jnp.float32),
        mesh=mesh,
        scratch_types=[
            pltpu.VMEM((nch, _DCH), jnp.int32),
            pltpu.VMEM((nch, _DCH), jnp.int32),
            pltpu.VMEM((_APW,), jnp.float32),
            pltpu.VMEM((_CCH, _HID), jnp.float32),
            pltpu.VMEM((_CCH, _HID), jnp.float32),
            pltpu.VMEM((_CCH, _HID), jnp.float32),
            pltpu.VMEM((_CCH, _HID), jnp.float32),
            pltpu.VMEM((_CCH, _HID), jnp.float32),
            pltpu.SemaphoreType.DMA,
            pltpu.SemaphoreType.DMA,
        ],
        compiler_params=pltpu.CompilerParams(needs_layout_passes=False),
    )
    return f(h, d0, d1, wt)


def kernel(x, Wproj, bproj, Wrouter, Wgate, Wup, Wdown):
    xp = x.reshape(_T, _PATCH)
    tokens, topi, topw, rank, counts8 = _router(
        xp, Wproj.T, bproj.reshape(1, _HID), Wrouter.T)

    counts = counts8[0]
    csum = jnp.cumsum(counts)
    offs = (csum - counts).astype(jnp.int32)
    ends = csum.astype(jnp.int32)

    ei = topi.reshape(_A)
    rk = rank.reshape(_A)
    wt = topw.reshape(_A)
    xs, d0, d1 = _dispatch(tokens, ei, rk, offs)

    t_start = offs // _TM
    t_end = (ends + _TM - 1) // _TM
    visits = jnp.where(counts > 0, t_end - t_start, 0)
    cumv = jnp.cumsum(visits)
    first_step = cumv - visits
    v_tot = cumv[-1]
    steps = jnp.arange(_S, dtype=jnp.int32)
    s_eff = jnp.minimum(steps, v_tot - 1)
    g_of = jnp.sum((s_eff[:, None] >= cumv[None, :]).astype(jnp.int32),
                   axis=1)
    t_of = (t_start[g_of] + (s_eff - first_step[g_of])).astype(jnp.int32)
    firsts = jnp.concatenate([
        jnp.ones((1,), jnp.int32),
        (t_of[1:] != t_of[:-1]).astype(jnp.int32),
    ])
    valid = (steps < v_tot).astype(jnp.int32)

    h = _ffn(g_of.astype(jnp.int32), t_of, firsts, offs, ends, valid,
             xs, Wgate, Wup, Wdown)
    outf = _combine(h, d0, d1, wt)
    return outf.reshape(_B, _C * _P, _HID)
